# Initial kernel scaffold; baseline (speedup 1.0000x reference)
import jax
import jax.numpy as jnp
from jax.experimental import pallas as pl


def kernel(inputs, W_in, b_in, embeddings, W_out, b_out):
    raise NotImplementedError("write your pallas kernel here")


# speedup vs baseline: 1.2110x; 1.2110x over previous
import functools

import jax
import jax.numpy as jnp
from jax.experimental import pallas as pl
from jax.experimental.pallas import tpu as pltpu

EMBED_DIM = 256
CODEBOOK = 8192
INPUT_DIM = 768
N_ROWS = 16 * 576
TILE = 256
N_TILES = N_ROWS // TILE


def _vq_kernel(inp_ref, w_in_ref, b_in_ref, emb_ref, w_out_ref, b_out_ref,
               out_ref, enc_ref, idx_ref, loss_ref, emb_n_ref):
    i = pl.program_id(0)

    @pl.when(i == 0)
    def _init():
        emb = emb_ref[...]
        norms = jnp.sqrt(jnp.sum(emb * emb, axis=0, keepdims=True))
        emb_n_ref[...] = emb / (norms + 1e-12)
        loss_ref[...] = jnp.zeros((1, 1), jnp.float32)

    x = jnp.dot(inp_ref[...].astype(jnp.bfloat16),
                w_in_ref[...].astype(jnp.bfloat16),
                preferred_element_type=jnp.float32) + b_in_ref[...]
    norm = jnp.sqrt(jnp.sum(x * x, axis=1, keepdims=True))
    x_n = x / (norm + 1e-12)
    logits = jnp.dot(x_n.astype(jnp.bfloat16),
                     emb_n_ref[...].astype(jnp.bfloat16),
                     preferred_element_type=jnp.float32)

    SLAB = CODEBOOK // 4
    iota_s = jax.lax.broadcasted_iota(jnp.int32, (TILE, SLAB), 1)

    def slab_argmax(s):
        sl = logits[:, s * SLAB:(s + 1) * SLAB]
        m = jnp.max(sl, axis=1, keepdims=True)
        j = jnp.min(jnp.where(sl == m, iota_s, CODEBOOK), axis=1) + s * SLAB
        return m[:, 0], j

    accv, accj = slab_argmax(0)
    accv = accv.astype(jnp.bfloat16).astype(jnp.float32)
    for s in range(1, 4):
        v, j = slab_argmax(s)
        upd = (v > accv) | ((v == accv) & (j < accj))
        accv = jnp.where(upd, v.astype(jnp.bfloat16).astype(jnp.float32), accv)
        accj = jnp.where(upd, j, accj)
    idx = accj
    iota = jax.lax.broadcasted_iota(jnp.int32, (TILE, CODEBOOK), 1)

    onehot = (iota == idx[:, None]).astype(jnp.float32)
    enc_ref[...] = onehot
    idx_ref[0, 0, :] = idx

    q = jax.lax.dot_general(
        onehot.astype(jnp.bfloat16), emb_ref[...].astype(jnp.bfloat16),
        (((1,), (1,)), ((), ())), preferred_element_type=jnp.float32)

    out_ref[...] = jnp.dot(q.astype(jnp.bfloat16),
                           w_out_ref[...].astype(jnp.bfloat16),
                           preferred_element_type=jnp.float32) + b_out_ref[...]
    diff = q - x
    part = 2.0 * jnp.sum(diff * diff) / (N_ROWS * EMBED_DIM)
    loss_ref[...] += part.reshape(1, 1)


@functools.partial(jax.jit, static_argnames=("interpret",))
def kernel(inputs, W_in, b_in, embeddings, W_out, b_out, interpret=False):
    B, T, _ = inputs.shape
    inp_flat = inputs.reshape(N_ROWS, INPUT_DIM)

    out, enc, idx, loss = pl.pallas_call(
        _vq_kernel,
        grid=(N_TILES,),
        in_specs=[
            pl.BlockSpec((TILE, INPUT_DIM), lambda i: (i, 0)),
            pl.BlockSpec((INPUT_DIM, EMBED_DIM), lambda i: (0, 0)),
            pl.BlockSpec((1, EMBED_DIM), lambda i: (0, 0)),
            pl.BlockSpec((EMBED_DIM, CODEBOOK), lambda i: (0, 0)),
            pl.BlockSpec((EMBED_DIM, INPUT_DIM), lambda i: (0, 0)),
            pl.BlockSpec((1, INPUT_DIM), lambda i: (0, 0)),
        ],
        out_specs=[
            pl.BlockSpec((TILE, INPUT_DIM), lambda i: (i, 0)),
            pl.BlockSpec((TILE, CODEBOOK), lambda i: (i, 0)),
            pl.BlockSpec((1, 1, TILE), lambda i: (i, 0, 0)),
            pl.BlockSpec((1, 1), lambda i: (0, 0)),
        ],
        out_shape=[
            jax.ShapeDtypeStruct((N_ROWS, INPUT_DIM), jnp.float32),
            jax.ShapeDtypeStruct((N_ROWS, CODEBOOK), jnp.float32),
            jax.ShapeDtypeStruct((N_TILES, 1, TILE), jnp.int32),
            jax.ShapeDtypeStruct((1, 1), jnp.float32),
        ],
        scratch_shapes=[pltpu.VMEM((EMBED_DIM, CODEBOOK), jnp.float32)],
        interpret=interpret,
    )(inp_flat, W_in, b_in.reshape(1, EMBED_DIM), embeddings,
      W_out, b_out.reshape(1, INPUT_DIM))

    encoding_indices = idx.reshape(B, T)
    return (out.reshape(B, T, INPUT_DIM), enc, encoding_indices, loss[0, 0])
